# Initial kernel scaffold; baseline (speedup 1.0000x reference)
import jax
import jax.numpy as jnp
from jax.experimental import pallas as pl


def kernel(x, edge_index, mode, W_lin1, W_seq1, w_f11, b_f11, w_f21, b_f21, bias1, W_lin2, W_seq2, w_f12, b_f12, w_f22, b_f22, bias2, Wc, bc):
    raise NotImplementedError("write your pallas kernel here")


# speedup vs baseline: 5.6466x; 5.6466x over previous
import functools

import jax
import jax.numpy as jnp
from jax import lax
from jax.experimental import pallas as pl
from jax.experimental.pallas import tpu as pltpu
from jax.experimental.pallas import tpu_sc as plsc

N = 10000
E = 320000
DIN = 128
FR = 128
HID = 64
NCLS = 16
ALPHA = 0.2

NC = 2
NS = 16
NW = NC * NS
L = 16

EPT = 10240
E_PAD = EPT * NW
CH = 128
NCH = EPT // CH
NPAD = 10240
RPT = NPAD // NS

BLK = 2000


def _dense_in_body(x_ref, wl_ref, ws_ref, wf_ref, bv_ref, seq_ref, f12_ref):
    xp = jnp.dot(x_ref[...], wl_ref[...], preferred_element_type=jnp.float32)
    seq = jnp.dot(xp, ws_ref[...], preferred_element_type=jnp.float32)
    seq_ref[...] = seq
    f12_ref[...] = (
        jnp.dot(seq, wf_ref[...], preferred_element_type=jnp.float32)
        + bv_ref[...]
    )


def _dense_in(x, wlt, wst, wf, bv):
    n, din = x.shape
    grid = (n // BLK,)
    return pl.pallas_call(
        _dense_in_body,
        grid=grid,
        in_specs=[
            pl.BlockSpec((BLK, din), lambda i: (i, 0)),
            pl.BlockSpec(wlt.shape, lambda i: (0, 0)),
            pl.BlockSpec(wst.shape, lambda i: (0, 0)),
            pl.BlockSpec(wf.shape, lambda i: (0, 0)),
            pl.BlockSpec(bv.shape, lambda i: (0, 0)),
        ],
        out_specs=[
            pl.BlockSpec((BLK, HID), lambda i: (i, 0)),
            pl.BlockSpec((BLK, 2), lambda i: (i, 0)),
        ],
        out_shape=[
            jax.ShapeDtypeStruct((n, HID), jnp.float32),
            jax.ShapeDtypeStruct((n, 2), jnp.float32),
        ],
    )(x, wlt, wst, wf, bv)


def _norm_dense_body(acc_ref, cs_ref, b_ref, wl_ref, ws_ref, wf_ref, bv_ref,
                     seq_ref, f12_ref):
    a = acc_ref[0] + acc_ref[1]
    s = cs_ref[:, 0:1] + cs_ref[:, 1:2]
    h = a / s + b_ref[...]
    xp = jnp.dot(h, wl_ref[...], preferred_element_type=jnp.float32)
    seq = jnp.dot(xp, ws_ref[...], preferred_element_type=jnp.float32)
    seq_ref[...] = seq
    f12_ref[...] = (
        jnp.dot(seq, wf_ref[...], preferred_element_type=jnp.float32)
        + bv_ref[...]
    )


def _norm_dense(acc, csn, b, wlt, wst, wf, bv):
    n = acc.shape[1]
    grid = (n // BLK,)
    return pl.pallas_call(
        _norm_dense_body,
        grid=grid,
        in_specs=[
            pl.BlockSpec((2, BLK, HID), lambda i: (0, i, 0)),
            pl.BlockSpec((BLK, 2), lambda i: (i, 0)),
            pl.BlockSpec(b.shape, lambda i: (0, 0)),
            pl.BlockSpec(wlt.shape, lambda i: (0, 0)),
            pl.BlockSpec(wst.shape, lambda i: (0, 0)),
            pl.BlockSpec(wf.shape, lambda i: (0, 0)),
            pl.BlockSpec(bv.shape, lambda i: (0, 0)),
        ],
        out_specs=[
            pl.BlockSpec((BLK, HID), lambda i: (i, 0)),
            pl.BlockSpec((BLK, 2), lambda i: (i, 0)),
        ],
        out_shape=[
            jax.ShapeDtypeStruct((n, HID), jnp.float32),
            jax.ShapeDtypeStruct((n, 2), jnp.float32),
        ],
    )(acc, csn, b, wlt, wst, wf, bv)


def _norm_out_body(acc_ref, cs_ref, b_ref, wc_ref, bc_ref, h_ref, out_ref):
    a = acc_ref[0] + acc_ref[1]
    s = cs_ref[:, 0:1] + cs_ref[:, 1:2]
    h = a / s + b_ref[...]
    h_ref[...] = h
    out_ref[...] = (
        jnp.dot(h, wc_ref[...], preferred_element_type=jnp.float32)
        + bc_ref[...]
    )


def _norm_out(acc, csn, b, wct, bc):
    n = acc.shape[1]
    grid = (n // BLK,)
    return pl.pallas_call(
        _norm_out_body,
        grid=grid,
        in_specs=[
            pl.BlockSpec((2, BLK, HID), lambda i: (0, i, 0)),
            pl.BlockSpec((BLK, 2), lambda i: (i, 0)),
            pl.BlockSpec(b.shape, lambda i: (0, 0)),
            pl.BlockSpec(wct.shape, lambda i: (0, 0)),
            pl.BlockSpec(bc.shape, lambda i: (0, 0)),
        ],
        out_specs=[
            pl.BlockSpec((BLK, HID), lambda i: (i, 0)),
            pl.BlockSpec((BLK, NCLS), lambda i: (i, 0)),
        ],
        out_shape=[
            jax.ShapeDtypeStruct((n, HID), jnp.float32),
            jax.ShapeDtypeStruct((n, NCLS), jnp.float32),
        ],
    )(acc, csn, b, wct, bc)


_SC_MESH = plsc.VectorSubcoreMesh(
    core_axis_name="c", subcore_axis_name="s", num_cores=NC, num_subcores=NS)


@functools.partial(
    pl.kernel,
    out_type=[
        jax.ShapeDtypeStruct((NC, NPAD, HID), jnp.float32),
        jax.ShapeDtypeStruct((NC, NPAD), jnp.float32),
    ],
    mesh=_SC_MESH,
    compiler_params=pltpu.CompilerParams(
        needs_layout_passes=False, use_tc_tiling_on_sc=False),
    scratch_types=[
        pltpu.VMEM((NCH, CH), jnp.int32),
        pltpu.VMEM((NCH, CH), jnp.int32),
        pltpu.VMEM((N,), jnp.float32),
        pltpu.VMEM((N,), jnp.float32),
        pltpu.VMEM((NCH, CH), jnp.float32),
        pltpu.VMEM((CH, HID), jnp.float32),
        pltpu.VMEM((RPT,), jnp.float32),
        pltpu.VMEM_SHARED((NPAD, HID), jnp.float32),
        pltpu.VMEM_SHARED((NPAD,), jnp.float32),
        pltpu.SemaphoreType.DMA,
    ],
)
def _edge_sc(src_hbm, dst_hbm, f12_hbm, seq_hbm, acc_hbm, cs_hbm,
             src_v, dst_v, f1_v, f2_v, coef_v, rows_v, cbuf_v,
             acc_sh, cs_sh, sem):
    cid = lax.axis_index("c")
    sid = lax.axis_index("s")
    wid = cid * NS + sid
    ebase = wid * EPT
    zk = jnp.zeros((L,), jnp.float32)
    iota = lax.iota(jnp.int32, L)

    pltpu.sync_copy(src_hbm.at[wid], src_v)
    pltpu.sync_copy(dst_hbm.at[wid], dst_v)
    pltpu.sync_copy(f12_hbm.at[0], f1_v)
    pltpu.sync_copy(f12_hbm.at[1], f2_v)

    def _zrow(r, carry):
        for c4 in range(HID // L):
            rows_v[r, pl.ds(c4 * L, L)] = zk
        return carry
    lax.fori_loop(0, CH, _zrow, 0)

    def _zcb(i, carry):
        cbuf_v[pl.ds(pl.multiple_of(i * L, L), L)] = zk
        return carry
    lax.fori_loop(0, RPT // L, _zcb, 0)

    rbase = pl.multiple_of(sid * RPT, CH)
    for j in range(RPT // CH):
        pltpu.sync_copy(rows_v, acc_sh.at[pl.ds(rbase + j * CH, CH)])
    pltpu.sync_copy(cbuf_v, cs_sh.at[pl.ds(rbase, RPT)])
    plsc.subcore_barrier()

    def _coef_chunk(ch, carry):
        for k in range(CH // L):
            s16 = src_v[ch, pl.ds(k * L, L)]
            d16 = dst_v[ch, pl.ds(k * L, L)]
            lo = plsc.load_gather(f1_v, [s16]) + plsc.load_gather(f2_v, [d16])
            lr = jnp.where(lo > 0, lo, jnp.float32(ALPHA) * lo)
            c = jnp.exp(lr)
            eid = ebase + ch * CH + k * L + iota
            c = jnp.where(eid < E, c, jnp.float32(0.0))
            coef_v[ch, pl.ds(k * L, L)] = c
        return carry
    lax.fori_loop(0, NCH, _coef_chunk, 0)

    def _scatter_chunk(ch, carry):
        pltpu.async_copy(seq_hbm.at[dst_v.at[ch]], rows_v, sem).wait()
        for k in range(CH // L):
            row_idx = k * L + iota
            c16 = coef_v[ch, pl.ds(k * L, L)]
            for col in range(HID):
                col_idx = jnp.full((L,), col, jnp.int32)
                v = plsc.load_gather(rows_v, [row_idx, col_idx])
                plsc.store_scatter(rows_v, [row_idx, col_idx], v * c16)
        pltpu.sync_copy(rows_v, acc_sh.at[src_v.at[ch]], add=True)
        pltpu.sync_copy(coef_v.at[ch], cs_sh.at[src_v.at[ch]], add=True)
        return carry
    lax.fori_loop(0, NCH, _scatter_chunk, 0)
    plsc.subcore_barrier()

    for j in range(RPT // CH):
        pltpu.sync_copy(acc_sh.at[pl.ds(rbase + j * CH, CH)], rows_v)
        pltpu.sync_copy(rows_v, acc_hbm.at[cid, pl.ds(rbase + j * CH, CH)])
    pltpu.sync_copy(cs_sh.at[pl.ds(rbase, RPT)], cbuf_v)
    pltpu.sync_copy(cbuf_v, cs_hbm.at[cid, pl.ds(rbase, RPT)])


def kernel(x, edge_index, mode, W_lin1, W_seq1, w_f11, b_f11, w_f21, b_f21,
           bias1, W_lin2, W_seq2, w_f12, b_f12, w_f22, b_f22, bias2, Wc, bc):
    src = edge_index[0]
    dst = edge_index[1]
    pad = jnp.zeros((E_PAD - E,), jnp.int32)
    src3 = jnp.concatenate([src, pad]).reshape(NW, NCH, CH)
    dst3 = jnp.concatenate([dst, pad]).reshape(NW, NCH, CH)

    wf1 = jnp.stack([w_f11, w_f21], axis=1)
    bv1 = jnp.stack([b_f11, b_f21])[None, :]
    seq1, f12_1 = _dense_in(x, W_lin1.T, W_seq1.T, wf1, bv1)

    acc1, cs1 = _edge_sc(src3, dst3, f12_1.T, seq1)

    wf2 = jnp.stack([w_f12, w_f22], axis=1)
    bv2 = jnp.stack([b_f12, b_f22])[None, :]
    seq2, f12_2 = _norm_dense(
        acc1[:, :N], jnp.transpose(cs1)[:N], bias1[None, :],
        W_lin2.T, W_seq2.T, wf2, bv2)

    acc2, cs2 = _edge_sc(src3, dst3, f12_2.T, seq2)

    h2, out = _norm_out(
        acc2[:, :N], jnp.transpose(cs2)[:N], bias2[None, :],
        Wc.T, bc[None, :])
    return (out, h2)


# speedup vs baseline: 18.9577x; 3.3574x over previous
import functools

import jax
import jax.numpy as jnp
from jax import lax
from jax.experimental import pallas as pl
from jax.experimental.pallas import tpu as pltpu
from jax.experimental.pallas import tpu_sc as plsc

N = 10000
E = 320000
DIN = 128
FR = 128
HID = 64
NCLS = 16
ALPHA = 0.2

NC = 2
NS = 16
NW = NC * NS
L = 16

EPT = 10240
E_PAD = EPT * NW
CH = 128
NCH = EPT // CH
NPAD = 10240
RPT = NPAD // NS

BLK = 2000


def _dense_in_body(x_ref, wl_ref, ws_ref, wf_ref, bv_ref, seq_ref, f12_ref):
    xp = jnp.dot(x_ref[...], wl_ref[...], preferred_element_type=jnp.float32)
    seq = jnp.dot(xp, ws_ref[...], preferred_element_type=jnp.float32)
    seq_ref[...] = seq
    f12_ref[...] = (
        jnp.dot(seq, wf_ref[...], preferred_element_type=jnp.float32)
        + bv_ref[...]
    )


def _dense_in(x, wlt, wst, wf, bv):
    n, din = x.shape
    grid = (n // BLK,)
    return pl.pallas_call(
        _dense_in_body,
        grid=grid,
        in_specs=[
            pl.BlockSpec((BLK, din), lambda i: (i, 0)),
            pl.BlockSpec(wlt.shape, lambda i: (0, 0)),
            pl.BlockSpec(wst.shape, lambda i: (0, 0)),
            pl.BlockSpec(wf.shape, lambda i: (0, 0)),
            pl.BlockSpec(bv.shape, lambda i: (0, 0)),
        ],
        out_specs=[
            pl.BlockSpec((BLK, HID), lambda i: (i, 0)),
            pl.BlockSpec((BLK, 2), lambda i: (i, 0)),
        ],
        out_shape=[
            jax.ShapeDtypeStruct((n, HID), jnp.float32),
            jax.ShapeDtypeStruct((n, 2), jnp.float32),
        ],
    )(x, wlt, wst, wf, bv)


def _norm_dense_body(acc_ref, cs_ref, b_ref, wl_ref, ws_ref, wf_ref, bv_ref,
                     seq_ref, f12_ref):
    a = acc_ref[0] + acc_ref[1]
    s = cs_ref[:, 0:1] + cs_ref[:, 1:2]
    h = a / s + b_ref[...]
    xp = jnp.dot(h, wl_ref[...], preferred_element_type=jnp.float32)
    seq = jnp.dot(xp, ws_ref[...], preferred_element_type=jnp.float32)
    seq_ref[...] = seq
    f12_ref[...] = (
        jnp.dot(seq, wf_ref[...], preferred_element_type=jnp.float32)
        + bv_ref[...]
    )


def _norm_dense(acc, csn, b, wlt, wst, wf, bv):
    n = acc.shape[1]
    grid = (n // BLK,)
    return pl.pallas_call(
        _norm_dense_body,
        grid=grid,
        in_specs=[
            pl.BlockSpec((2, BLK, HID), lambda i: (0, i, 0)),
            pl.BlockSpec((BLK, 2), lambda i: (i, 0)),
            pl.BlockSpec(b.shape, lambda i: (0, 0)),
            pl.BlockSpec(wlt.shape, lambda i: (0, 0)),
            pl.BlockSpec(wst.shape, lambda i: (0, 0)),
            pl.BlockSpec(wf.shape, lambda i: (0, 0)),
            pl.BlockSpec(bv.shape, lambda i: (0, 0)),
        ],
        out_specs=[
            pl.BlockSpec((BLK, HID), lambda i: (i, 0)),
            pl.BlockSpec((BLK, 2), lambda i: (i, 0)),
        ],
        out_shape=[
            jax.ShapeDtypeStruct((n, HID), jnp.float32),
            jax.ShapeDtypeStruct((n, 2), jnp.float32),
        ],
    )(acc, csn, b, wlt, wst, wf, bv)


def _norm_out_body(acc_ref, cs_ref, b_ref, wc_ref, bc_ref, h_ref, out_ref):
    a = acc_ref[0] + acc_ref[1]
    s = cs_ref[:, 0:1] + cs_ref[:, 1:2]
    h = a / s + b_ref[...]
    h_ref[...] = h
    out_ref[...] = (
        jnp.dot(h, wc_ref[...], preferred_element_type=jnp.float32)
        + bc_ref[...]
    )


def _norm_out(acc, csn, b, wct, bc):
    n = acc.shape[1]
    grid = (n // BLK,)
    return pl.pallas_call(
        _norm_out_body,
        grid=grid,
        in_specs=[
            pl.BlockSpec((2, BLK, HID), lambda i: (0, i, 0)),
            pl.BlockSpec((BLK, 2), lambda i: (i, 0)),
            pl.BlockSpec(b.shape, lambda i: (0, 0)),
            pl.BlockSpec(wct.shape, lambda i: (0, 0)),
            pl.BlockSpec(bc.shape, lambda i: (0, 0)),
        ],
        out_specs=[
            pl.BlockSpec((BLK, HID), lambda i: (i, 0)),
            pl.BlockSpec((BLK, NCLS), lambda i: (i, 0)),
        ],
        out_shape=[
            jax.ShapeDtypeStruct((n, HID), jnp.float32),
            jax.ShapeDtypeStruct((n, NCLS), jnp.float32),
        ],
    )(acc, csn, b, wct, bc)


_SC_MESH = plsc.VectorSubcoreMesh(
    core_axis_name="c", subcore_axis_name="s", num_cores=NC, num_subcores=NS)


@functools.partial(
    pl.kernel,
    out_type=[
        jax.ShapeDtypeStruct((NC, NPAD, HID), jnp.float32),
        jax.ShapeDtypeStruct((NC, NPAD), jnp.float32),
    ],
    mesh=_SC_MESH,
    compiler_params=pltpu.CompilerParams(
        needs_layout_passes=False, use_tc_tiling_on_sc=False),
    scratch_types=[
        pltpu.VMEM((NCH, CH), jnp.int32),
        pltpu.VMEM((NCH, CH), jnp.int32),
        pltpu.VMEM((N,), jnp.float32),
        pltpu.VMEM((N,), jnp.float32),
        pltpu.VMEM((NCH, CH), jnp.float32),
        pltpu.VMEM((CH, HID), jnp.float32),
        pltpu.VMEM((CH, HID), jnp.float32),
        pltpu.VMEM((CH, HID), jnp.float32),
        pltpu.VMEM((CH, HID), jnp.float32),
        pltpu.VMEM((RPT,), jnp.float32),
        pltpu.VMEM_SHARED((NPAD, HID), jnp.float32),
        pltpu.VMEM_SHARED((NPAD,), jnp.float32),
        pltpu.SemaphoreType.DMA,
        pltpu.SemaphoreType.DMA,
        pltpu.SemaphoreType.DMA,
        pltpu.SemaphoreType.DMA,
        pltpu.SemaphoreType.DMA,
        pltpu.SemaphoreType.DMA,
    ],
)
def _edge_sc(src_hbm, dst_hbm, f12_hbm, seq_hbm, acc_hbm, cs_hbm,
             src_v, dst_v, f1_v, f2_v, coef_v, rin0, rin1, rout0, rout1,
             cbuf_v, acc_sh, cs_sh, g0, g1, sa0, sa1, sc0, sc1):
    cid = lax.axis_index("c")
    sid = lax.axis_index("s")
    wid = cid * NS + sid
    ebase = wid * EPT
    zk = jnp.zeros((L,), jnp.float32)
    iota = lax.iota(jnp.int32, L)

    pltpu.sync_copy(src_hbm.at[wid], src_v)
    pltpu.sync_copy(dst_hbm.at[wid], dst_v)
    pltpu.sync_copy(f12_hbm.at[0], f1_v)
    pltpu.sync_copy(f12_hbm.at[1], f2_v)

    def _zrow(r, carry):
        for c4 in range(HID // L):
            rout0[r, pl.ds(c4 * L, L)] = zk
        return carry
    lax.fori_loop(0, CH, _zrow, 0)

    def _zcb(i, carry):
        cbuf_v[pl.ds(pl.multiple_of(i * L, L), L)] = zk
        return carry
    lax.fori_loop(0, RPT // L, _zcb, 0)

    rbase = pl.multiple_of(sid * RPT, CH)
    for j in range(RPT // CH):
        pltpu.sync_copy(rout0, acc_sh.at[pl.ds(rbase + j * CH, CH)])
    pltpu.sync_copy(cbuf_v, cs_sh.at[pl.ds(rbase, RPT)])
    plsc.subcore_barrier()

    pltpu.async_copy(seq_hbm.at[dst_v.at[0]], rin0, g0)

    def _coef_chunk(ch, carry):
        for k in range(CH // L):
            s16 = src_v[ch, pl.ds(k * L, L)]
            d16 = dst_v[ch, pl.ds(k * L, L)]
            lo = plsc.load_gather(f1_v, [s16]) + plsc.load_gather(f2_v, [d16])
            lr = jnp.where(lo > 0, lo, jnp.float32(ALPHA) * lo)
            c = jnp.exp(lr)
            eid = ebase + ch * CH + k * L + iota
            c = jnp.where(eid < E, c, jnp.float32(0.0))
            coef_v[ch, pl.ds(k * L, L)] = c
        return carry
    lax.fori_loop(0, NCH, _coef_chunk, 0)

    bufs = ((rin0, rout0, g0, sa0, sc0), (rin1, rout1, g1, sa1, sc1))

    splats = [jnp.full((L,), i, jnp.int32) for i in range(L)]

    def _scale(ch, rin, rout):
        def _kbody(k, carry):
            c16 = coef_v[ch, pl.ds(pl.multiple_of(k * L, L), L)]
            rowb = k * L
            for i in range(L):
                cb = c16.at[splats[i]].get(mode="promise_in_bounds")
                r = rowb + i
                for q in range(HID // L):
                    rout[r, pl.ds(q * L, L)] = rin[r, pl.ds(q * L, L)] * cb
            return carry
        lax.fori_loop(0, CH // L, _kbody, 0)

    def _slot(j, ch, p, issue_next):
        rin, rout, g, sa, sc = bufs[p]
        nrin, ng = bufs[1 - p][0], bufs[1 - p][2]
        pltpu.make_async_copy(seq_hbm.at[dst_v.at[ch]], rin, g).wait()

        if issue_next is None:
            pltpu.async_copy(seq_hbm.at[dst_v.at[ch + 1]], nrin, ng)
        else:
            @pl.when(issue_next)
            def _():
                pltpu.async_copy(seq_hbm.at[dst_v.at[ch + 1]], nrin, ng)

        @pl.when(j > 0)
        def _():
            pltpu.make_async_copy(rout, acc_sh.at[src_v.at[ch - 2]], sa).wait()
            pltpu.make_async_copy(
                coef_v.at[ch - 2], cs_sh.at[src_v.at[ch - 2]], sc).wait()

        _scale(ch, rin, rout)
        pltpu.async_copy(rout, acc_sh.at[src_v.at[ch]], sa, add=True)
        pltpu.async_copy(coef_v.at[ch], cs_sh.at[src_v.at[ch]], sc, add=True)

    def _pair(j, carry):
        _slot(j, 2 * j, 0, None)
        _slot(j, 2 * j + 1, 1, j < NCH // 2 - 1)
        return carry
    lax.fori_loop(0, NCH // 2, _pair, 0)

    pltpu.make_async_copy(rout0, acc_sh.at[src_v.at[NCH - 2]], sa0).wait()
    pltpu.make_async_copy(
        coef_v.at[NCH - 2], cs_sh.at[src_v.at[NCH - 2]], sc0).wait()
    pltpu.make_async_copy(rout1, acc_sh.at[src_v.at[NCH - 1]], sa1).wait()
    pltpu.make_async_copy(
        coef_v.at[NCH - 1], cs_sh.at[src_v.at[NCH - 1]], sc1).wait()
    plsc.subcore_barrier()

    for j in range(RPT // CH):
        pltpu.sync_copy(acc_sh.at[pl.ds(rbase + j * CH, CH)], rout0)
        pltpu.sync_copy(rout0, acc_hbm.at[cid, pl.ds(rbase + j * CH, CH)])
    pltpu.sync_copy(cs_sh.at[pl.ds(rbase, RPT)], cbuf_v)
    pltpu.sync_copy(cbuf_v, cs_hbm.at[cid, pl.ds(rbase, RPT)])


def kernel(x, edge_index, mode, W_lin1, W_seq1, w_f11, b_f11, w_f21, b_f21,
           bias1, W_lin2, W_seq2, w_f12, b_f12, w_f22, b_f22, bias2, Wc, bc):
    src = edge_index[0]
    dst = edge_index[1]
    pad = jnp.zeros((E_PAD - E,), jnp.int32)
    src3 = jnp.concatenate([src, pad]).reshape(NW, NCH, CH)
    dst3 = jnp.concatenate([dst, pad]).reshape(NW, NCH, CH)

    wf1 = jnp.stack([w_f11, w_f21], axis=1)
    bv1 = jnp.stack([b_f11, b_f21])[None, :]
    seq1, f12_1 = _dense_in(x, W_lin1.T, W_seq1.T, wf1, bv1)

    acc1, cs1 = _edge_sc(src3, dst3, f12_1.T, seq1)

    wf2 = jnp.stack([w_f12, w_f22], axis=1)
    bv2 = jnp.stack([b_f12, b_f22])[None, :]
    seq2, f12_2 = _norm_dense(
        acc1[:, :N], jnp.transpose(cs1)[:N], bias1[None, :],
        W_lin2.T, W_seq2.T, wf2, bv2)

    acc2, cs2 = _edge_sc(src3, dst3, f12_2.T, seq2)

    h2, out = _norm_out(
        acc2[:, :N], jnp.transpose(cs2)[:N], bias2[None, :],
        Wc.T, bc[None, :])
    return (out, h2)


# speedup vs baseline: 19.6970x; 1.0390x over previous
import functools

import jax
import jax.numpy as jnp
from jax import lax
from jax.experimental import pallas as pl
from jax.experimental.pallas import tpu as pltpu
from jax.experimental.pallas import tpu_sc as plsc

N = 10000
E = 320000
DIN = 128
FR = 128
HID = 64
NCLS = 16
ALPHA = 0.2

NC = 2
NS = 16
NW = NC * NS
L = 16

EPT = 10240
E_PAD = EPT * NW
CH = 128
NCH = EPT // CH
NPAD = 10240
RPT = NPAD // NS

BLK = 2000


def _dense_in_body(x_ref, wl_ref, ws_ref, wf_ref, bv_ref, seq_ref, f12_ref):
    xp = jnp.dot(x_ref[...], wl_ref[...], preferred_element_type=jnp.float32)
    seq = jnp.dot(xp, ws_ref[...], preferred_element_type=jnp.float32)
    seq_ref[...] = seq
    f12_ref[...] = (
        jnp.dot(seq, wf_ref[...], preferred_element_type=jnp.float32)
        + bv_ref[...]
    )


def _dense_in(x, wlt, wst, wf, bv):
    n, din = x.shape
    grid = (n // BLK,)
    return pl.pallas_call(
        _dense_in_body,
        grid=grid,
        in_specs=[
            pl.BlockSpec((BLK, din), lambda i: (i, 0)),
            pl.BlockSpec(wlt.shape, lambda i: (0, 0)),
            pl.BlockSpec(wst.shape, lambda i: (0, 0)),
            pl.BlockSpec(wf.shape, lambda i: (0, 0)),
            pl.BlockSpec(bv.shape, lambda i: (0, 0)),
        ],
        out_specs=[
            pl.BlockSpec((BLK, HID), lambda i: (i, 0)),
            pl.BlockSpec((BLK, 2), lambda i: (i, 0)),
        ],
        out_shape=[
            jax.ShapeDtypeStruct((n, HID), jnp.float32),
            jax.ShapeDtypeStruct((n, 2), jnp.float32),
        ],
    )(x, wlt, wst, wf, bv)


def _norm_dense_body(acc_ref, cs_ref, b_ref, wl_ref, ws_ref, wf_ref, bv_ref,
                     seq_ref, f12_ref):
    a = acc_ref[0] + acc_ref[1]
    s = cs_ref[:, 0:1] + cs_ref[:, 1:2]
    h = a / s + b_ref[...]
    xp = jnp.dot(h, wl_ref[...], preferred_element_type=jnp.float32)
    seq = jnp.dot(xp, ws_ref[...], preferred_element_type=jnp.float32)
    seq_ref[...] = seq
    f12_ref[...] = (
        jnp.dot(seq, wf_ref[...], preferred_element_type=jnp.float32)
        + bv_ref[...]
    )


def _norm_dense(acc, csn, b, wlt, wst, wf, bv):
    n = N
    grid = (n // BLK,)
    return pl.pallas_call(
        _norm_dense_body,
        grid=grid,
        in_specs=[
            pl.BlockSpec((2, BLK, HID), lambda i: (0, i, 0)),
            pl.BlockSpec((BLK, 2), lambda i: (i, 0)),
            pl.BlockSpec(b.shape, lambda i: (0, 0)),
            pl.BlockSpec(wlt.shape, lambda i: (0, 0)),
            pl.BlockSpec(wst.shape, lambda i: (0, 0)),
            pl.BlockSpec(wf.shape, lambda i: (0, 0)),
            pl.BlockSpec(bv.shape, lambda i: (0, 0)),
        ],
        out_specs=[
            pl.BlockSpec((BLK, HID), lambda i: (i, 0)),
            pl.BlockSpec((BLK, 2), lambda i: (i, 0)),
        ],
        out_shape=[
            jax.ShapeDtypeStruct((n, HID), jnp.float32),
            jax.ShapeDtypeStruct((n, 2), jnp.float32),
        ],
    )(acc, csn, b, wlt, wst, wf, bv)


def _norm_out_body(acc_ref, cs_ref, b_ref, wc_ref, bc_ref, h_ref, out_ref):
    a = acc_ref[0] + acc_ref[1]
    s = cs_ref[:, 0:1] + cs_ref[:, 1:2]
    h = a / s + b_ref[...]
    h_ref[...] = h
    out_ref[...] = (
        jnp.dot(h, wc_ref[...], preferred_element_type=jnp.float32)
        + bc_ref[...]
    )


def _norm_out(acc, csn, b, wct, bc):
    n = N
    grid = (n // BLK,)
    return pl.pallas_call(
        _norm_out_body,
        grid=grid,
        in_specs=[
            pl.BlockSpec((2, BLK, HID), lambda i: (0, i, 0)),
            pl.BlockSpec((BLK, 2), lambda i: (i, 0)),
            pl.BlockSpec(b.shape, lambda i: (0, 0)),
            pl.BlockSpec(wct.shape, lambda i: (0, 0)),
            pl.BlockSpec(bc.shape, lambda i: (0, 0)),
        ],
        out_specs=[
            pl.BlockSpec((BLK, HID), lambda i: (i, 0)),
            pl.BlockSpec((BLK, NCLS), lambda i: (i, 0)),
        ],
        out_shape=[
            jax.ShapeDtypeStruct((n, HID), jnp.float32),
            jax.ShapeDtypeStruct((n, NCLS), jnp.float32),
        ],
    )(acc, csn, b, wct, bc)


_SC_MESH = plsc.VectorSubcoreMesh(
    core_axis_name="c", subcore_axis_name="s", num_cores=NC, num_subcores=NS)


@functools.partial(
    pl.kernel,
    out_type=[
        jax.ShapeDtypeStruct((NC, NPAD, HID), jnp.float32),
        jax.ShapeDtypeStruct((NC, NPAD), jnp.float32),
    ],
    mesh=_SC_MESH,
    compiler_params=pltpu.CompilerParams(
        needs_layout_passes=False, use_tc_tiling_on_sc=False),
    scratch_types=[
        pltpu.VMEM((NCH, CH), jnp.int32),
        pltpu.VMEM((NCH, CH), jnp.int32),
        pltpu.VMEM((N,), jnp.float32),
        pltpu.VMEM((N,), jnp.float32),
        pltpu.VMEM((NCH, CH), jnp.float32),
        pltpu.VMEM((CH, HID), jnp.float32),
        pltpu.VMEM((CH, HID), jnp.float32),
        pltpu.VMEM((CH, HID), jnp.float32),
        pltpu.VMEM((CH, HID), jnp.float32),
        pltpu.VMEM((RPT,), jnp.float32),
        pltpu.VMEM_SHARED((NPAD, HID), jnp.float32),
        pltpu.VMEM_SHARED((NPAD,), jnp.float32),
        pltpu.SemaphoreType.DMA,
        pltpu.SemaphoreType.DMA,
        pltpu.SemaphoreType.DMA,
        pltpu.SemaphoreType.DMA,
        pltpu.SemaphoreType.DMA,
        pltpu.SemaphoreType.DMA,
    ],
)
def _edge_sc(src_hbm, dst_hbm, f12_hbm, seq_hbm, acc_hbm, cs_hbm,
             src_v, dst_v, f1_v, f2_v, coef_v, rin0, rin1, rout0, rout1,
             cbuf_v, acc_sh, cs_sh, g0, g1, sa0, sa1, sc0, sc1):
    cid = lax.axis_index("c")
    sid = lax.axis_index("s")
    wid = cid * NS + sid
    ebase = wid * EPT
    zk = jnp.zeros((L,), jnp.float32)
    iota = lax.iota(jnp.int32, L)

    pltpu.sync_copy(src_hbm.at[wid], src_v)
    pltpu.sync_copy(dst_hbm.at[wid], dst_v)
    pltpu.sync_copy(f12_hbm.at[0], f1_v)
    pltpu.sync_copy(f12_hbm.at[1], f2_v)

    def _zrow(r, carry):
        for c4 in range(HID // L):
            rout0[r, pl.ds(c4 * L, L)] = zk
        return carry
    lax.fori_loop(0, CH, _zrow, 0)

    def _zcb(i, carry):
        cbuf_v[pl.ds(pl.multiple_of(i * L, L), L)] = zk
        return carry
    lax.fori_loop(0, RPT // L, _zcb, 0)

    rbase = pl.multiple_of(sid * RPT, CH)
    for j in range(RPT // CH):
        pltpu.sync_copy(rout0, acc_sh.at[pl.ds(rbase + j * CH, CH)])
    pltpu.sync_copy(cbuf_v, cs_sh.at[pl.ds(rbase, RPT)])
    plsc.subcore_barrier()

    pltpu.async_copy(seq_hbm.at[dst_v.at[0]], rin0, g0)

    def _coef_chunk(ch, carry):
        for k in range(CH // L):
            s16 = src_v[ch, pl.ds(k * L, L)]
            d16 = dst_v[ch, pl.ds(k * L, L)]
            lo = plsc.load_gather(f1_v, [s16]) + plsc.load_gather(f2_v, [d16])
            lr = jnp.where(lo > 0, lo, jnp.float32(ALPHA) * lo)
            c = jnp.exp(lr)
            eid = ebase + ch * CH + k * L + iota
            c = jnp.where(eid < E, c, jnp.float32(0.0))
            coef_v[ch, pl.ds(k * L, L)] = c
        return carry
    lax.fori_loop(0, NCH, _coef_chunk, 0)

    bufs = ((rin0, rout0, g0, sa0, sc0), (rin1, rout1, g1, sa1, sc1))

    splats = [jnp.full((L,), i, jnp.int32) for i in range(L)]

    def _scale(ch, rin, rout):
        def _kbody(k, carry):
            c16 = coef_v[ch, pl.ds(pl.multiple_of(k * L, L), L)]
            rowb = k * L
            for i in range(L):
                cb = c16.at[splats[i]].get(mode="promise_in_bounds")
                r = rowb + i
                for q in range(HID // L):
                    rout[r, pl.ds(q * L, L)] = rin[r, pl.ds(q * L, L)] * cb
            return carry
        lax.fori_loop(0, CH // L, _kbody, 0)

    def _slot(j, ch, p, issue_next):
        rin, rout, g, sa, sc = bufs[p]
        nrin, ng = bufs[1 - p][0], bufs[1 - p][2]
        pltpu.make_async_copy(seq_hbm.at[dst_v.at[ch]], rin, g).wait()

        if issue_next is None:
            pltpu.async_copy(seq_hbm.at[dst_v.at[ch + 1]], nrin, ng)
        else:
            @pl.when(issue_next)
            def _():
                pltpu.async_copy(seq_hbm.at[dst_v.at[ch + 1]], nrin, ng)

        @pl.when(j > 0)
        def _():
            pltpu.make_async_copy(rout, acc_sh.at[src_v.at[ch - 2]], sa).wait()
            pltpu.make_async_copy(
                coef_v.at[ch - 2], cs_sh.at[src_v.at[ch - 2]], sc).wait()

        _scale(ch, rin, rout)
        pltpu.async_copy(rout, acc_sh.at[src_v.at[ch]], sa, add=True)
        pltpu.async_copy(coef_v.at[ch], cs_sh.at[src_v.at[ch]], sc, add=True)

    def _pair(j, carry):
        _slot(j, 2 * j, 0, None)
        _slot(j, 2 * j + 1, 1, j < NCH // 2 - 1)
        return carry
    lax.fori_loop(0, NCH // 2, _pair, 0)

    pltpu.make_async_copy(rout0, acc_sh.at[src_v.at[NCH - 2]], sa0).wait()
    pltpu.make_async_copy(
        coef_v.at[NCH - 2], cs_sh.at[src_v.at[NCH - 2]], sc0).wait()
    pltpu.make_async_copy(rout1, acc_sh.at[src_v.at[NCH - 1]], sa1).wait()
    pltpu.make_async_copy(
        coef_v.at[NCH - 1], cs_sh.at[src_v.at[NCH - 1]], sc1).wait()
    plsc.subcore_barrier()

    for j in range(RPT // CH):
        pltpu.sync_copy(acc_sh.at[pl.ds(rbase + j * CH, CH)], rout0)
        pltpu.sync_copy(rout0, acc_hbm.at[cid, pl.ds(rbase + j * CH, CH)])
    pltpu.sync_copy(cs_sh.at[pl.ds(rbase, RPT)], cbuf_v)
    pltpu.sync_copy(cbuf_v, cs_hbm.at[cid, pl.ds(rbase, RPT)])


def kernel(x, edge_index, mode, W_lin1, W_seq1, w_f11, b_f11, w_f21, b_f21,
           bias1, W_lin2, W_seq2, w_f12, b_f12, w_f22, b_f22, bias2, Wc, bc):
    src = edge_index[0]
    dst = edge_index[1]
    pad_src = N + (jnp.arange(E_PAD - E, dtype=jnp.int32) % (NPAD - N))
    pad_dst = jnp.zeros((E_PAD - E,), jnp.int32)
    src3 = jnp.concatenate([src, pad_src]).reshape(NW, NCH, CH)
    dst3 = jnp.concatenate([dst, pad_dst]).reshape(NW, NCH, CH)

    wf1 = jnp.stack([w_f11, w_f21], axis=1)
    bv1 = jnp.stack([b_f11, b_f21])[None, :]
    seq1, f12_1 = _dense_in(x, W_lin1.T, W_seq1.T, wf1, bv1)

    acc1, cs1 = _edge_sc(src3, dst3, f12_1.T, seq1)

    wf2 = jnp.stack([w_f12, w_f22], axis=1)
    bv2 = jnp.stack([b_f12, b_f22])[None, :]
    seq2, f12_2 = _norm_dense(
        acc1, jnp.transpose(cs1), bias1[None, :],
        W_lin2.T, W_seq2.T, wf2, bv2)

    acc2, cs2 = _edge_sc(src3, dst3, f12_2.T, seq2)

    h2, out = _norm_out(
        acc2, jnp.transpose(cs2), bias2[None, :],
        Wc.T, bc[None, :])
    return (out, h2)


# speedup vs baseline: 24.8229x; 1.2602x over previous
import functools

import jax
import jax.numpy as jnp
import numpy as np
from jax import lax
from jax.experimental import pallas as pl
from jax.experimental.pallas import tpu as pltpu
from jax.experimental.pallas import tpu_sc as plsc

N = 10000
E = 320000
DIN = 128
FR = 128
HID = 64
NCLS = 16
ALPHA = 0.2

NC = 2
NS = 16
NW = NC * NS
L = 16

EPT = 10240
E_PAD = EPT * NW
CH = 128
NCH = EPT // CH
NPAD = 10240
RPT = NPAD // NS

BLK = 2000

_PERM = np.empty(HID, dtype=np.int32)
for _q in range(HID // 32):
    for _j in range(16):
        _PERM[32 * _q + 2 * _j] = 32 * _q + _j
        _PERM[32 * _q + 2 * _j + 1] = 32 * _q + 16 + _j
_PM = np.zeros((HID, HID), np.float32)
for _c in range(HID):
    _PM[_PERM[_c], _c] = 1.0


def _dense_in_body(x_ref, wl_ref, ws_ref, wf_ref, bv_ref, seq_ref, f12_ref):
    xp = jnp.dot(x_ref[...], wl_ref[...], preferred_element_type=jnp.float32)
    seq = jnp.dot(xp, ws_ref[...], preferred_element_type=jnp.float32)
    seq_ref[...] = seq.astype(jnp.bfloat16)
    f12_ref[...] = (
        jnp.dot(seq, wf_ref[...], preferred_element_type=jnp.float32)
        + bv_ref[...]
    )


def _dense_in(x, wlt, wst, wf, bv):
    n, din = x.shape
    grid = (n // BLK,)
    return pl.pallas_call(
        _dense_in_body,
        grid=grid,
        in_specs=[
            pl.BlockSpec((BLK, din), lambda i: (i, 0)),
            pl.BlockSpec(wlt.shape, lambda i: (0, 0)),
            pl.BlockSpec(wst.shape, lambda i: (0, 0)),
            pl.BlockSpec(wf.shape, lambda i: (0, 0)),
            pl.BlockSpec(bv.shape, lambda i: (0, 0)),
        ],
        out_specs=[
            pl.BlockSpec((BLK, HID), lambda i: (i, 0)),
            pl.BlockSpec((BLK, 2), lambda i: (i, 0)),
        ],
        out_shape=[
            jax.ShapeDtypeStruct((n, HID), jnp.bfloat16),
            jax.ShapeDtypeStruct((n, 2), jnp.float32),
        ],
    )(x, wlt, wst, wf, bv)


def _norm_dense_body(acc_ref, cs_ref, pm_ref, b_ref, wl_ref, ws_ref, wf_ref,
                     bv_ref, seq_ref, f12_ref):
    a = jnp.dot(acc_ref[0] + acc_ref[1], pm_ref[...],
                preferred_element_type=jnp.float32)
    s = cs_ref[:, 0:1] + cs_ref[:, 1:2]
    h = a / s + b_ref[...]
    xp = jnp.dot(h, wl_ref[...], preferred_element_type=jnp.float32)
    seq = jnp.dot(xp, ws_ref[...], preferred_element_type=jnp.float32)
    seq_ref[...] = seq.astype(jnp.bfloat16)
    f12_ref[...] = (
        jnp.dot(seq, wf_ref[...], preferred_element_type=jnp.float32)
        + bv_ref[...]
    )


def _norm_dense(acc, csn, pm, b, wlt, wst, wf, bv):
    n = N
    grid = (n // BLK,)
    return pl.pallas_call(
        _norm_dense_body,
        grid=grid,
        in_specs=[
            pl.BlockSpec((2, BLK, HID), lambda i: (0, i, 0)),
            pl.BlockSpec((BLK, 2), lambda i: (i, 0)),
            pl.BlockSpec(pm.shape, lambda i: (0, 0)),
            pl.BlockSpec(b.shape, lambda i: (0, 0)),
            pl.BlockSpec(wlt.shape, lambda i: (0, 0)),
            pl.BlockSpec(wst.shape, lambda i: (0, 0)),
            pl.BlockSpec(wf.shape, lambda i: (0, 0)),
            pl.BlockSpec(bv.shape, lambda i: (0, 0)),
        ],
        out_specs=[
            pl.BlockSpec((BLK, HID), lambda i: (i, 0)),
            pl.BlockSpec((BLK, 2), lambda i: (i, 0)),
        ],
        out_shape=[
            jax.ShapeDtypeStruct((n, HID), jnp.bfloat16),
            jax.ShapeDtypeStruct((n, 2), jnp.float32),
        ],
    )(acc, csn, pm, b, wlt, wst, wf, bv)


def _norm_out_body(acc_ref, cs_ref, pm_ref, b_ref, wc_ref, bc_ref, h_ref,
                   out_ref):
    a = jnp.dot(acc_ref[0] + acc_ref[1], pm_ref[...],
                preferred_element_type=jnp.float32)
    s = cs_ref[:, 0:1] + cs_ref[:, 1:2]
    h = a / s + b_ref[...]
    h_ref[...] = h
    out_ref[...] = (
        jnp.dot(h, wc_ref[...], preferred_element_type=jnp.float32)
        + bc_ref[...]
    )


def _norm_out(acc, csn, pm, b, wct, bc):
    n = N
    grid = (n // BLK,)
    return pl.pallas_call(
        _norm_out_body,
        grid=grid,
        in_specs=[
            pl.BlockSpec((2, BLK, HID), lambda i: (0, i, 0)),
            pl.BlockSpec((BLK, 2), lambda i: (i, 0)),
            pl.BlockSpec(pm.shape, lambda i: (0, 0)),
            pl.BlockSpec(b.shape, lambda i: (0, 0)),
            pl.BlockSpec(wct.shape, lambda i: (0, 0)),
            pl.BlockSpec(bc.shape, lambda i: (0, 0)),
        ],
        out_specs=[
            pl.BlockSpec((BLK, HID), lambda i: (i, 0)),
            pl.BlockSpec((BLK, NCLS), lambda i: (i, 0)),
        ],
        out_shape=[
            jax.ShapeDtypeStruct((n, HID), jnp.float32),
            jax.ShapeDtypeStruct((n, NCLS), jnp.float32),
        ],
    )(acc, csn, pm, b, wct, bc)


_SC_MESH = plsc.VectorSubcoreMesh(
    core_axis_name="c", subcore_axis_name="s", num_cores=NC, num_subcores=NS)


@functools.partial(
    pl.kernel,
    out_type=[
        jax.ShapeDtypeStruct((NC, NPAD, HID), jnp.float32),
        jax.ShapeDtypeStruct((NC, NPAD), jnp.float32),
    ],
    mesh=_SC_MESH,
    compiler_params=pltpu.CompilerParams(
        needs_layout_passes=False, use_tc_tiling_on_sc=False),
    scratch_types=[
        pltpu.VMEM((NCH, CH), jnp.int32),
        pltpu.VMEM((NCH, CH), jnp.int32),
        pltpu.VMEM((N,), jnp.float32),
        pltpu.VMEM((N,), jnp.float32),
        pltpu.VMEM((NCH, CH), jnp.float32),
        pltpu.VMEM((CH, HID // 2), jnp.int32),
        pltpu.VMEM((CH, HID // 2), jnp.int32),
        pltpu.VMEM((CH, HID), jnp.float32),
        pltpu.VMEM((CH, HID), jnp.float32),
        pltpu.VMEM((RPT,), jnp.float32),
        pltpu.VMEM_SHARED((NPAD, HID), jnp.float32),
        pltpu.VMEM_SHARED((NPAD,), jnp.float32),
        pltpu.SemaphoreType.DMA,
        pltpu.SemaphoreType.DMA,
        pltpu.SemaphoreType.DMA,
        pltpu.SemaphoreType.DMA,
        pltpu.SemaphoreType.DMA,
        pltpu.SemaphoreType.DMA,
    ],
)
def _edge_sc(src_hbm, dst_hbm, f12_hbm, seq_hbm, acc_hbm, cs_hbm,
             src_v, dst_v, f1_v, f2_v, coef_v, rin0, rin1, rout0, rout1,
             cbuf_v, acc_sh, cs_sh, g0, g1, sa0, sa1, sc0, sc1):
    cid = lax.axis_index("c")
    sid = lax.axis_index("s")
    wid = cid * NS + sid
    ebase = wid * EPT
    zk = jnp.zeros((L,), jnp.float32)
    iota = lax.iota(jnp.int32, L)

    pltpu.sync_copy(src_hbm.at[wid], src_v)
    pltpu.sync_copy(dst_hbm.at[wid], dst_v)
    pltpu.sync_copy(f12_hbm.at[0], f1_v)
    pltpu.sync_copy(f12_hbm.at[1], f2_v)

    def _zrow(r, carry):
        for c4 in range(HID // L):
            rout0[r, pl.ds(c4 * L, L)] = zk
        return carry
    lax.fori_loop(0, CH, _zrow, 0)

    def _zcb(i, carry):
        cbuf_v[pl.ds(pl.multiple_of(i * L, L), L)] = zk
        return carry
    lax.fori_loop(0, RPT // L, _zcb, 0)

    rbase = pl.multiple_of(sid * RPT, CH)
    for j in range(RPT // CH):
        pltpu.sync_copy(rout0, acc_sh.at[pl.ds(rbase + j * CH, CH)])
    pltpu.sync_copy(cbuf_v, cs_sh.at[pl.ds(rbase, RPT)])

    def _coef_chunk(ch, carry):
        for k in range(CH // L):
            s16 = src_v[ch, pl.ds(k * L, L)]
            d16 = dst_v[ch, pl.ds(k * L, L)]
            lo = plsc.load_gather(f1_v, [s16]) + plsc.load_gather(f2_v, [d16])
            lr = jnp.where(lo > 0, lo, jnp.float32(ALPHA) * lo)
            c = jnp.exp(lr)
            eid = ebase + ch * CH + k * L + iota
            c = jnp.where(eid < E, c, jnp.float32(0.0))
            coef_v[ch, pl.ds(k * L, L)] = c
        return carry
    lax.fori_loop(0, NCH, _coef_chunk, 0)

    plsc.subcore_barrier()

    pltpu.async_copy(seq_hbm.at[dst_v.at[0]], rin0, g0)

    bufs = ((rin0, rout0, g0, sa0, sc0), (rin1, rout1, g1, sa1, sc1))

    splats = [jnp.full((L,), i, jnp.int32) for i in range(L)]

    def _scale(ch, rin, rout):
        hmask = jnp.full((L,), -65536, jnp.int32)

        def _kbody(k, carry):
            c16 = coef_v[ch, pl.ds(pl.multiple_of(k * L, L), L)]
            rowb = k * L
            for i in range(L):
                cb = c16.at[splats[i]].get(mode="promise_in_bounds")
                r = rowb + i
                for q in range(HID // 32):
                    v = rin[r, pl.ds(q * L, L)]
                    ev = lax.bitcast_convert_type(v << 16, jnp.float32)
                    od = lax.bitcast_convert_type(v & hmask, jnp.float32)
                    rout[r, pl.ds(q * 32, L)] = ev * cb
                    rout[r, pl.ds(q * 32 + L, L)] = od * cb
            return carry
        lax.fori_loop(0, CH // L, _kbody, 0)

    def _slot(j, ch, p, issue_next):
        rin, rout, g, sa, sc = bufs[p]
        nrin, ng = bufs[1 - p][0], bufs[1 - p][2]
        pltpu.make_async_copy(seq_hbm.at[dst_v.at[ch]], rin, g).wait()

        if issue_next is None:
            pltpu.async_copy(seq_hbm.at[dst_v.at[ch + 1]], nrin, ng)
        else:
            @pl.when(issue_next)
            def _():
                pltpu.async_copy(seq_hbm.at[dst_v.at[ch + 1]], nrin, ng)

        @pl.when(j > 0)
        def _():
            pltpu.make_async_copy(rout, acc_sh.at[src_v.at[ch - 2]], sa).wait()
            pltpu.make_async_copy(
                coef_v.at[ch - 2], cs_sh.at[src_v.at[ch - 2]], sc).wait()

        _scale(ch, rin, rout)
        pltpu.async_copy(rout, acc_sh.at[src_v.at[ch]], sa, add=True)
        pltpu.async_copy(coef_v.at[ch], cs_sh.at[src_v.at[ch]], sc, add=True)

    def _pair(j, carry):
        _slot(j, 2 * j, 0, None)
        _slot(j, 2 * j + 1, 1, j < NCH // 2 - 1)
        return carry
    lax.fori_loop(0, NCH // 2, _pair, 0)

    pltpu.make_async_copy(rout0, acc_sh.at[src_v.at[NCH - 2]], sa0).wait()
    pltpu.make_async_copy(
        coef_v.at[NCH - 2], cs_sh.at[src_v.at[NCH - 2]], sc0).wait()
    pltpu.make_async_copy(rout1, acc_sh.at[src_v.at[NCH - 1]], sa1).wait()
    pltpu.make_async_copy(
        coef_v.at[NCH - 1], cs_sh.at[src_v.at[NCH - 1]], sc1).wait()
    plsc.subcore_barrier()

    for j in range(RPT // CH):
        pltpu.sync_copy(acc_sh.at[pl.ds(rbase + j * CH, CH)], rout0)
        pltpu.sync_copy(rout0, acc_hbm.at[cid, pl.ds(rbase + j * CH, CH)])
    pltpu.sync_copy(cs_sh.at[pl.ds(rbase, RPT)], cbuf_v)
    pltpu.sync_copy(cbuf_v, cs_hbm.at[cid, pl.ds(rbase, RPT)])


def kernel(x, edge_index, mode, W_lin1, W_seq1, w_f11, b_f11, w_f21, b_f21,
           bias1, W_lin2, W_seq2, w_f12, b_f12, w_f22, b_f22, bias2, Wc, bc):
    src = edge_index[0]
    dst = edge_index[1]
    pad_src = N + (jnp.arange(E_PAD - E, dtype=jnp.int32) % (NPAD - N))
    pad_dst = jnp.zeros((E_PAD - E,), jnp.int32)
    src3 = jnp.concatenate([src, pad_src]).reshape(NW, NCH, CH)
    dst3 = jnp.concatenate([dst, pad_dst]).reshape(NW, NCH, CH)

    pm = jnp.asarray(_PM)

    def _packed(seqb):
        return lax.bitcast_convert_type(
            seqb.reshape(N, HID // 2, 2), jnp.int32)

    wf1 = jnp.stack([w_f11, w_f21], axis=1)
    bv1 = jnp.stack([b_f11, b_f21])[None, :]
    seqb1, f12_1 = _dense_in(x, W_lin1.T, W_seq1.T, wf1, bv1)

    acc1, cs1 = _edge_sc(src3, dst3, f12_1.T, _packed(seqb1))

    wf2 = jnp.stack([w_f12, w_f22], axis=1)
    bv2 = jnp.stack([b_f12, b_f22])[None, :]
    seqb2, f12_2 = _norm_dense(
        acc1, jnp.transpose(cs1), pm, bias1[None, :],
        W_lin2.T, W_seq2.T, wf2, bv2)

    acc2, cs2 = _edge_sc(src3, dst3, f12_2.T, _packed(seqb2))

    h2, out = _norm_out(
        acc2, jnp.transpose(cs2), pm, bias2[None, :],
        Wc.T, bc[None, :])
    return (out, h2)


# speedup vs baseline: 27.5760x; 1.1109x over previous
import functools

import jax
import jax.numpy as jnp
import numpy as np
from jax import lax
from jax.experimental import pallas as pl
from jax.experimental.pallas import tpu as pltpu
from jax.experimental.pallas import tpu_sc as plsc

N = 10000
E = 320000
DIN = 128
FR = 128
HID = 64
NCLS = 16
ALPHA = 0.2

NC = 2
NS = 16
NW = NC * NS
L = 16

EPT = 10240
E_PAD = EPT * NW
CH = 128
NCH = EPT // CH
NPAD = 10240
RPT = NPAD // NS

BLK = 2000

_PERM = np.empty(HID, dtype=np.int32)
for _q in range(HID // 32):
    for _j in range(16):
        _PERM[32 * _q + 2 * _j] = 32 * _q + _j
        _PERM[32 * _q + 2 * _j + 1] = 32 * _q + 16 + _j
_PM = np.zeros((HID, HID), np.float32)
for _c in range(HID):
    _PM[_PERM[_c], _c] = 1.0


def _dense_in_body(x_ref, wl_ref, ws_ref, wf_ref, bv_ref, seq_ref, f12_ref):
    xp = jnp.dot(x_ref[...], wl_ref[...], preferred_element_type=jnp.float32)
    seq = jnp.dot(xp, ws_ref[...], preferred_element_type=jnp.float32)
    seq_ref[...] = seq.astype(jnp.bfloat16)
    f12_ref[...] = (
        jnp.dot(seq, wf_ref[...], preferred_element_type=jnp.float32)
        + bv_ref[...]
    )


def _dense_in(x, wlt, wst, wf, bv):
    n, din = x.shape
    grid = (n // BLK,)
    return pl.pallas_call(
        _dense_in_body,
        grid=grid,
        in_specs=[
            pl.BlockSpec((BLK, din), lambda i: (i, 0)),
            pl.BlockSpec(wlt.shape, lambda i: (0, 0)),
            pl.BlockSpec(wst.shape, lambda i: (0, 0)),
            pl.BlockSpec(wf.shape, lambda i: (0, 0)),
            pl.BlockSpec(bv.shape, lambda i: (0, 0)),
        ],
        out_specs=[
            pl.BlockSpec((BLK, HID), lambda i: (i, 0)),
            pl.BlockSpec((BLK, 2), lambda i: (i, 0)),
        ],
        out_shape=[
            jax.ShapeDtypeStruct((n, HID), jnp.bfloat16),
            jax.ShapeDtypeStruct((n, 2), jnp.float32),
        ],
    )(x, wlt, wst, wf, bv)


def _norm_dense_body(acc_ref, cs_ref, pm_ref, b_ref, wl_ref, ws_ref, wf_ref,
                     bv_ref, seq_ref, f12_ref):
    a = jnp.dot(acc_ref[0] + acc_ref[1], pm_ref[...],
                preferred_element_type=jnp.float32)
    s = cs_ref[:, 0:1] + cs_ref[:, 1:2]
    h = a / s + b_ref[...]
    xp = jnp.dot(h, wl_ref[...], preferred_element_type=jnp.float32)
    seq = jnp.dot(xp, ws_ref[...], preferred_element_type=jnp.float32)
    seq_ref[...] = seq.astype(jnp.bfloat16)
    f12_ref[...] = (
        jnp.dot(seq, wf_ref[...], preferred_element_type=jnp.float32)
        + bv_ref[...]
    )


def _norm_dense(acc, csn, pm, b, wlt, wst, wf, bv):
    n = N
    grid = (n // BLK,)
    return pl.pallas_call(
        _norm_dense_body,
        grid=grid,
        in_specs=[
            pl.BlockSpec((2, BLK, HID), lambda i: (0, i, 0)),
            pl.BlockSpec((BLK, 2), lambda i: (i, 0)),
            pl.BlockSpec(pm.shape, lambda i: (0, 0)),
            pl.BlockSpec(b.shape, lambda i: (0, 0)),
            pl.BlockSpec(wlt.shape, lambda i: (0, 0)),
            pl.BlockSpec(wst.shape, lambda i: (0, 0)),
            pl.BlockSpec(wf.shape, lambda i: (0, 0)),
            pl.BlockSpec(bv.shape, lambda i: (0, 0)),
        ],
        out_specs=[
            pl.BlockSpec((BLK, HID), lambda i: (i, 0)),
            pl.BlockSpec((BLK, 2), lambda i: (i, 0)),
        ],
        out_shape=[
            jax.ShapeDtypeStruct((n, HID), jnp.bfloat16),
            jax.ShapeDtypeStruct((n, 2), jnp.float32),
        ],
    )(acc, csn, pm, b, wlt, wst, wf, bv)


def _norm_out_body(acc_ref, cs_ref, pm_ref, b_ref, wc_ref, bc_ref, h_ref,
                   out_ref):
    a = jnp.dot(acc_ref[0] + acc_ref[1], pm_ref[...],
                preferred_element_type=jnp.float32)
    s = cs_ref[:, 0:1] + cs_ref[:, 1:2]
    h = a / s + b_ref[...]
    h_ref[...] = h
    out_ref[...] = (
        jnp.dot(h, wc_ref[...], preferred_element_type=jnp.float32)
        + bc_ref[...]
    )


def _norm_out(acc, csn, pm, b, wct, bc):
    n = N
    grid = (n // BLK,)
    return pl.pallas_call(
        _norm_out_body,
        grid=grid,
        in_specs=[
            pl.BlockSpec((2, BLK, HID), lambda i: (0, i, 0)),
            pl.BlockSpec((BLK, 2), lambda i: (i, 0)),
            pl.BlockSpec(pm.shape, lambda i: (0, 0)),
            pl.BlockSpec(b.shape, lambda i: (0, 0)),
            pl.BlockSpec(wct.shape, lambda i: (0, 0)),
            pl.BlockSpec(bc.shape, lambda i: (0, 0)),
        ],
        out_specs=[
            pl.BlockSpec((BLK, HID), lambda i: (i, 0)),
            pl.BlockSpec((BLK, NCLS), lambda i: (i, 0)),
        ],
        out_shape=[
            jax.ShapeDtypeStruct((n, HID), jnp.float32),
            jax.ShapeDtypeStruct((n, NCLS), jnp.float32),
        ],
    )(acc, csn, pm, b, wct, bc)


_SC_MESH = plsc.VectorSubcoreMesh(
    core_axis_name="c", subcore_axis_name="s", num_cores=NC, num_subcores=NS)


@functools.partial(
    pl.kernel,
    out_type=[
        jax.ShapeDtypeStruct((NC, NPAD, HID), jnp.float32),
        jax.ShapeDtypeStruct((NC, NPAD), jnp.float32),
    ],
    mesh=_SC_MESH,
    compiler_params=pltpu.CompilerParams(
        needs_layout_passes=False, use_tc_tiling_on_sc=False),
    scratch_types=[
        pltpu.VMEM((NCH, CH), jnp.int32),
        pltpu.VMEM((NCH, CH), jnp.int32),
        pltpu.VMEM((N,), jnp.float32),
        pltpu.VMEM((N,), jnp.float32),
        pltpu.VMEM((NCH, CH), jnp.float32),
        pltpu.VMEM((CH, HID // 2), jnp.int32),
        pltpu.VMEM((CH, HID // 2), jnp.int32),
        pltpu.VMEM((CH, HID), jnp.float32),
        pltpu.VMEM((CH, HID), jnp.float32),
        pltpu.VMEM((RPT,), jnp.float32),
        pltpu.VMEM_SHARED((NPAD, HID), jnp.float32),
        pltpu.VMEM_SHARED((NPAD,), jnp.float32),
        pltpu.SemaphoreType.DMA,
        pltpu.SemaphoreType.DMA,
        pltpu.SemaphoreType.DMA,
        pltpu.SemaphoreType.DMA,
        pltpu.SemaphoreType.DMA,
        pltpu.SemaphoreType.DMA,
    ],
)
def _edge_sc(src_hbm, dst_hbm, f12_hbm, seq_hbm, acc_hbm, cs_hbm,
             src_v, dst_v, f1_v, f2_v, coef_v, rin0, rin1, rout0, rout1,
             cbuf_v, acc_sh, cs_sh, g0, g1, sa0, sa1, sc0, sc1):
    cid = lax.axis_index("c")
    sid = lax.axis_index("s")
    wid = cid * NS + sid
    ebase = wid * EPT
    zk = jnp.zeros((L,), jnp.float32)
    iota = lax.iota(jnp.int32, L)

    pltpu.sync_copy(src_hbm.at[wid], src_v)
    pltpu.sync_copy(dst_hbm.at[wid], dst_v)
    pltpu.sync_copy(f12_hbm.at[0], f1_v)
    pltpu.sync_copy(f12_hbm.at[1], f2_v)

    def _zrow(r, carry):
        for c4 in range(HID // L):
            rout0[r, pl.ds(c4 * L, L)] = zk
        return carry
    lax.fori_loop(0, CH, _zrow, 0)

    def _zcb(i, carry):
        cbuf_v[pl.ds(pl.multiple_of(i * L, L), L)] = zk
        return carry
    lax.fori_loop(0, RPT // L, _zcb, 0)

    rbase = pl.multiple_of(sid * RPT, CH)
    for j in range(RPT // CH):
        pltpu.sync_copy(rout0, acc_sh.at[pl.ds(rbase + j * CH, CH)])
    pltpu.sync_copy(cbuf_v, cs_sh.at[pl.ds(rbase, RPT)])

    def _coef_one(ch):
        for k in range(CH // L):
            s16 = src_v[ch, pl.ds(k * L, L)]
            d16 = dst_v[ch, pl.ds(k * L, L)]
            lo = plsc.load_gather(f1_v, [s16]) + plsc.load_gather(f2_v, [d16])
            lr = jnp.where(lo > 0, lo, jnp.float32(ALPHA) * lo)
            c = jnp.exp(lr)
            eid = ebase + ch * CH + k * L + iota
            c = jnp.where(eid < E, c, jnp.float32(0.0))
            coef_v[ch, pl.ds(k * L, L)] = c

    pltpu.async_copy(seq_hbm.at[dst_v.at[0]], rin0, g0)
    _coef_one(0)
    _coef_one(1)

    plsc.subcore_barrier()

    bufs = ((rin0, rout0, g0, sa0, sc0), (rin1, rout1, g1, sa1, sc1))

    splats = [jnp.full((L,), i, jnp.int32) for i in range(L)]

    def _scale(ch, rin, rout):
        hmask = jnp.full((L,), -65536, jnp.int32)

        def _kbody(k, carry):
            c16 = coef_v[ch, pl.ds(pl.multiple_of(k * L, L), L)]
            rowb = k * L
            for i in range(L):
                cb = c16.at[splats[i]].get(mode="promise_in_bounds")
                r = rowb + i
                for q in range(HID // 32):
                    v = rin[r, pl.ds(q * L, L)]
                    ev = lax.bitcast_convert_type(v << 16, jnp.float32)
                    od = lax.bitcast_convert_type(v & hmask, jnp.float32)
                    rout[r, pl.ds(q * 32, L)] = ev * cb
                    rout[r, pl.ds(q * 32 + L, L)] = od * cb
            return carry
        lax.fori_loop(0, CH // L, _kbody, 0)

    def _slot(j, ch, p, issue_next, cnext):
        rin, rout, g, sa, sc = bufs[p]
        nrin, ng = bufs[1 - p][0], bufs[1 - p][2]

        if issue_next is None:
            pltpu.async_copy(seq_hbm.at[dst_v.at[ch + 1]], nrin, ng)
        else:
            @pl.when(issue_next)
            def _():
                pltpu.async_copy(seq_hbm.at[dst_v.at[ch + 1]], nrin, ng)

        @pl.when(cnext)
        def _():
            _coef_one(ch + 2)

        pltpu.make_async_copy(seq_hbm.at[dst_v.at[ch]], rin, g).wait()

        @pl.when(j > 0)
        def _():
            pltpu.make_async_copy(rout, acc_sh.at[src_v.at[ch - 2]], sa).wait()
            pltpu.make_async_copy(
                coef_v.at[ch - 2], cs_sh.at[src_v.at[ch - 2]], sc).wait()

        _scale(ch, rin, rout)
        pltpu.async_copy(rout, acc_sh.at[src_v.at[ch]], sa, add=True)
        pltpu.async_copy(coef_v.at[ch], cs_sh.at[src_v.at[ch]], sc, add=True)

    def _pair(j, carry):
        nx = j < NCH // 2 - 1
        _slot(j, 2 * j, 0, None, nx)
        _slot(j, 2 * j + 1, 1, nx, nx)
        return carry
    lax.fori_loop(0, NCH // 2, _pair, 0)

    pltpu.make_async_copy(rout0, acc_sh.at[src_v.at[NCH - 2]], sa0).wait()
    pltpu.make_async_copy(
        coef_v.at[NCH - 2], cs_sh.at[src_v.at[NCH - 2]], sc0).wait()
    pltpu.make_async_copy(rout1, acc_sh.at[src_v.at[NCH - 1]], sa1).wait()
    pltpu.make_async_copy(
        coef_v.at[NCH - 1], cs_sh.at[src_v.at[NCH - 1]], sc1).wait()
    plsc.subcore_barrier()

    for j in range(RPT // CH):
        pltpu.sync_copy(acc_sh.at[pl.ds(rbase + j * CH, CH)], rout0)
        pltpu.sync_copy(rout0, acc_hbm.at[cid, pl.ds(rbase + j * CH, CH)])
    pltpu.sync_copy(cs_sh.at[pl.ds(rbase, RPT)], cbuf_v)
    pltpu.sync_copy(cbuf_v, cs_hbm.at[cid, pl.ds(rbase, RPT)])


def kernel(x, edge_index, mode, W_lin1, W_seq1, w_f11, b_f11, w_f21, b_f21,
           bias1, W_lin2, W_seq2, w_f12, b_f12, w_f22, b_f22, bias2, Wc, bc):
    src = edge_index[0]
    dst = edge_index[1]
    pad_src = N + (jnp.arange(E_PAD - E, dtype=jnp.int32) % (NPAD - N))
    pad_dst = jnp.zeros((E_PAD - E,), jnp.int32)
    src3 = jnp.concatenate([src, pad_src]).reshape(NW, NCH, CH)
    dst3 = jnp.concatenate([dst, pad_dst]).reshape(NW, NCH, CH)

    pm = jnp.asarray(_PM)

    def _packed(seqb):
        return lax.bitcast_convert_type(
            seqb.reshape(N, HID // 2, 2), jnp.int32)

    wf1 = jnp.stack([w_f11, w_f21], axis=1)
    bv1 = jnp.stack([b_f11, b_f21])[None, :]
    seqb1, f12_1 = _dense_in(x, W_lin1.T, W_seq1.T, wf1, bv1)

    acc1, cs1 = _edge_sc(src3, dst3, f12_1.T, _packed(seqb1))

    wf2 = jnp.stack([w_f12, w_f22], axis=1)
    bv2 = jnp.stack([b_f12, b_f22])[None, :]
    seqb2, f12_2 = _norm_dense(
        acc1, jnp.transpose(cs1), pm, bias1[None, :],
        W_lin2.T, W_seq2.T, wf2, bv2)

    acc2, cs2 = _edge_sc(src3, dst3, f12_2.T, _packed(seqb2))

    h2, out = _norm_out(
        acc2, jnp.transpose(cs2), pm, bias2[None, :],
        Wc.T, bc[None, :])
    return (out, h2)
